# baseline (speedup 1.0000x reference)
import functools

import jax
import jax.numpy as jnp
from jax.experimental import pallas as pl
from jax.experimental.pallas import tpu as pltpu


def _gem_body(x_ref, o_ref, acc_ref, *, eps, inv_cols, inv_p, n_k):
    k = pl.program_id(1)

    @pl.when(k == 0)
    def _():
        acc_ref[...] = jnp.zeros_like(acc_ref)

    total = acc_ref[...]
    for j in range(x_ref.shape[0]):
        v = jnp.maximum(x_ref[j], eps)
        total += v * v * v
    acc_ref[...] = total

    @pl.when(k == n_k - 1)
    def _():
        m = acc_ref[...] * inv_cols
        y2 = jnp.exp(jnp.log(m) * inv_p)
        o_ref[...] = y2.reshape(o_ref.shape)


@functools.partial(jax.jit, static_argnames=("p", "eps"))
def _gem_pool(x, p=3.0, eps=1e-6):
    N, C, H, W = x.shape
    HW = H * W

    xt = x.transpose(2, 3, 0, 1).reshape(HW, N, C)

    TN = N // 2
    KB = 7
    n_k = pl.cdiv(HW, KB)
    grid = (pl.cdiv(N, TN), n_k)

    body = functools.partial(
        _gem_body, eps=float(eps), inv_cols=1.0 / float(HW),
        inv_p=1.0 / float(p), n_k=n_k)

    y = pl.pallas_call(
        body,
        out_shape=jax.ShapeDtypeStruct((N, C // 128, 128), jnp.float32),
        grid=grid,
        in_specs=[pl.BlockSpec((KB, TN, C), lambda i, k: (k, i, 0))],
        out_specs=pl.BlockSpec((TN, C // 128, 128), lambda i, k: (i, 0, 0)),
        scratch_shapes=[pltpu.VMEM((TN, C), jnp.float32)],
        compiler_params=pltpu.CompilerParams(
            dimension_semantics=("parallel", "arbitrary")),
    )(xt)

    return y.astype(x.dtype).reshape(N, C, 1, 1)


def kernel(x):
    return _gem_pool(x, 3.0, eps=1e-6)


# speedup vs baseline: 1.2687x; 1.2687x over previous
import functools

import jax
import jax.numpy as jnp
from jax.experimental import pallas as pl
from jax.experimental.pallas import tpu as pltpu


def _gem_body(xa_ref, xb_ref, o_ref, acc_ref, *, eps, inv_cols, inv_p, n_k):
    k = pl.program_id(0)
    hn = xa_ref.shape[1]

    @pl.when(k == 0)
    def _():
        acc_ref[...] = jnp.zeros_like(acc_ref)

    ta = acc_ref[:hn]
    tb = acc_ref[hn:]
    for j in range(xa_ref.shape[0]):
        va = jnp.maximum(xa_ref[j], eps)
        vb = jnp.maximum(xb_ref[j], eps)
        ta += va * va * va
        tb += vb * vb * vb
    acc_ref[:hn] = ta
    acc_ref[hn:] = tb

    @pl.when(k == n_k - 1)
    def _():
        m = acc_ref[...] * inv_cols
        y2 = jnp.exp(jnp.log(m) * inv_p)
        o_ref[...] = y2.reshape(o_ref.shape)


@functools.partial(jax.jit, static_argnames=("p", "eps"))
def _gem_pool(x, p=3.0, eps=1e-6):
    N, C, H, W = x.shape
    HW = H * W

    xt = x.transpose(2, 3, 0, 1).reshape(HW, N, C)

    HN = N // 2
    KB = 7
    n_k = pl.cdiv(HW, KB)
    grid = (n_k,)

    body = functools.partial(
        _gem_body, eps=float(eps), inv_cols=1.0 / float(HW),
        inv_p=1.0 / float(p), n_k=n_k)

    y = pl.pallas_call(
        body,
        out_shape=jax.ShapeDtypeStruct((N, C // 128, 128), jnp.float32),
        grid=grid,
        in_specs=[
            pl.BlockSpec((KB, HN, C), lambda k: (k, 0, 0)),
            pl.BlockSpec((KB, HN, C), lambda k: (k, 1, 0)),
        ],
        out_specs=pl.BlockSpec((N, C // 128, 128), lambda k: (0, 0, 0)),
        scratch_shapes=[pltpu.VMEM((N, C), jnp.float32)],
        compiler_params=pltpu.CompilerParams(
            dimension_semantics=("arbitrary",)),
    )(xt, xt)

    return y.astype(x.dtype).reshape(N, C, 1, 1)


def kernel(x):
    return _gem_pool(x, 3.0, eps=1e-6)


# speedup vs baseline: 1.2983x; 1.0233x over previous
import functools

import jax
import jax.numpy as jnp
from jax.experimental import pallas as pl
from jax.experimental.pallas import tpu as pltpu


def _gem_body(x_ref, o_ref, acc_ref, *, eps, inv_cols, inv_p, n_k):
    k = pl.program_id(1)

    @pl.when(k == 0)
    def _():
        acc_ref[...] = jnp.zeros_like(acc_ref)

    total = acc_ref[...]
    for j in range(x_ref.shape[0]):
        v = jnp.maximum(x_ref[j], eps)
        total += v * v * v
    acc_ref[...] = total

    @pl.when(k == n_k - 1)
    def _():
        m = acc_ref[...] * inv_cols
        y2 = jnp.exp(jnp.log(m) * inv_p)
        o_ref[...] = y2.reshape(o_ref.shape)


@functools.partial(jax.jit, static_argnames=("p", "eps"))
def _gem_pool(x, p=3.0, eps=1e-6):
    N, C, H, W = x.shape
    HW = H * W

    xt = x.transpose(2, 3, 0, 1).reshape(HW, N, C)

    TN = N
    KB = 7
    n_k = pl.cdiv(HW, KB)
    grid = (pl.cdiv(N, TN), n_k)

    body = functools.partial(
        _gem_body, eps=float(eps), inv_cols=1.0 / float(HW),
        inv_p=1.0 / float(p), n_k=n_k)

    y = pl.pallas_call(
        body,
        out_shape=jax.ShapeDtypeStruct((N, C // 128, 128), jnp.float32),
        grid=grid,
        in_specs=[pl.BlockSpec((KB, TN, C), lambda i, k: (k, i, 0))],
        out_specs=pl.BlockSpec((TN, C // 128, 128), lambda i, k: (i, 0, 0)),
        scratch_shapes=[pltpu.VMEM((TN, C), jnp.float32)],
        compiler_params=pltpu.CompilerParams(
            dimension_semantics=("parallel", "arbitrary")),
    )(xt)

    return y.astype(x.dtype).reshape(N, C, 1, 1)


def kernel(x):
    return _gem_pool(x, 3.0, eps=1e-6)


# speedup vs baseline: 1.3027x; 1.0034x over previous
import functools

import jax
import jax.numpy as jnp
from jax.experimental import pallas as pl
from jax.experimental.pallas import tpu as pltpu


def _gem_body(x_ref, o_ref, acc_ref, *, eps, inv_cols, inv_p, n_k):
    k = pl.program_id(0)

    @pl.when(k == 0)
    def _():
        acc_ref[...] = jnp.zeros_like(acc_ref)

    total = acc_ref[...]
    for j in range(x_ref.shape[0]):
        v = jnp.maximum(x_ref[j], eps)
        total += v * v * v
    acc_ref[...] = total

    @pl.when(k == n_k - 1)
    def _():
        m = acc_ref[...] * inv_cols
        y2 = jnp.exp(jnp.log(m) * inv_p)
        o_ref[...] = y2.reshape(o_ref.shape)


@functools.partial(jax.jit, static_argnames=("p", "eps"))
def _gem_pool(x, p=3.0, eps=1e-6):
    N, C, H, W = x.shape
    HW = H * W

    xt = x.transpose(2, 3, 0, 1).reshape(HW, N, C)

    KB = 7
    n_k = pl.cdiv(HW, KB)

    body = functools.partial(
        _gem_body, eps=float(eps), inv_cols=1.0 / float(HW),
        inv_p=1.0 / float(p), n_k=n_k)

    y = pl.pallas_call(
        body,
        out_shape=jax.ShapeDtypeStruct((N, C // 128, 128), jnp.float32),
        grid=(n_k,),
        in_specs=[pl.BlockSpec((KB, N, C), lambda k: (k, 0, 0))],
        out_specs=pl.BlockSpec((N, C // 128, 128), lambda k: (0, 0, 0)),
        scratch_shapes=[pltpu.VMEM((N, C), jnp.float32)],
        compiler_params=pltpu.CompilerParams(
            dimension_semantics=("arbitrary",)),
    )(xt)

    return y.astype(x.dtype).reshape(N, C, 1, 1)


def kernel(x):
    return _gem_pool(x, 3.0, eps=1e-6)
